# baseline (speedup 1.0000x reference)
import numpy as np
import jax
import jax.numpy as jnp
from jax.experimental import pallas as pl
from jax.experimental.pallas import tpu as pltpu

B, H, W, C = 4, 32, 32, 192
N = H * W
IMG = 512
SIGMA = 4.0
KS = 33
_NB = 32
_NSTREAM = 4

_HIGH = jax.lax.Precision.HIGHEST


def _resize_blur_matrix():
    out_size, in_size = IMG, H
    s = (np.arange(out_size, dtype=np.float64) + 0.5) * (in_size / out_size) - 0.5
    j = np.arange(in_size, dtype=np.float64)
    w = np.maximum(0.0, 1.0 - np.abs(s[None, :] - j[:, None]))
    w /= w.sum(axis=0, keepdims=True)
    R = w.T
    x = np.arange(KS, dtype=np.float64) - (KS - 1) / 2.0
    g = np.exp(-(x ** 2) / (2.0 * SIGMA * SIGMA))
    g /= g.sum()
    pad = KS // 2
    Bl = np.zeros((out_size, out_size), dtype=np.float64)
    for i in range(out_size):
        for t in range(KS):
            src = i - pad + t
            if src < 0:
                src = -src
            elif src >= out_size:
                src = 2 * out_size - 2 - src
            Bl[i, src] += g[t]
    return (Bl @ R).astype(np.float32)


_A = _resize_blur_matrix()


def _maha_kernel(f_ref, m_ref, *c_and_o):
    c_refs, o_ref = c_and_o[:-1], c_and_o[-1]
    delta = f_ref[...] - m_ref[...]
    d2s = []
    for s, c_ref in enumerate(c_refs):
        dl = delta[s * _NB:(s + 1) * _NB]
        y = jax.lax.dot_general(
            dl, c_ref[...],
            dimension_numbers=(((2,), (1,)), ((0,), (0,))),
            precision=jax.lax.Precision.DEFAULT,
        )
        d2s.append(jnp.sum(y * dl, axis=2))
    d2 = jnp.concatenate(d2s, axis=0)
    o_ref[...] = jnp.sqrt(jnp.maximum(d2, 0.0))


def _mask_kernel(d_ref, a_ref, at_ref, mask_ref, score_ref):
    a = a_ref[...]
    at = at_ref[...]
    scores = []
    for b in range(B):
        t = jax.lax.dot(a, d_ref[b], precision=_HIGH)
        m = jax.lax.dot(t, at, precision=_HIGH)
        mask_ref[b] = m
        scores.append(jnp.max(m))
    score_ref[...] = jnp.stack(scores).reshape(B, 1)


def kernel(inputs, mean, cvar_inv):
    feature = inputs.reshape(B, N, C).transpose(1, 0, 2)
    mean3 = mean.reshape(N, 1, C)

    step = _NSTREAM * _NB
    cvar_specs = [
        pl.BlockSpec((_NB, C, C), lambda i, s=s: (_NSTREAM * i + s, 0, 0))
        for s in range(_NSTREAM)
    ]
    dist_nb = pl.pallas_call(
        _maha_kernel,
        grid=(N // step,),
        in_specs=[
            pl.BlockSpec((step, B, C), lambda i: (i, 0, 0)),
            pl.BlockSpec((step, 1, C), lambda i: (i, 0, 0)),
            *cvar_specs,
        ],
        out_specs=pl.BlockSpec((step, B), lambda i: (i, 0)),
        out_shape=jax.ShapeDtypeStruct((N, B), jnp.float32),
        compiler_params=pltpu.CompilerParams(
            dimension_semantics=("arbitrary",),
        ),
    )(feature, mean3, *([cvar_inv] * _NSTREAM))

    dist = dist_nb.T.reshape(B, H, W)
    a = jnp.asarray(_A)
    mask, score = pl.pallas_call(
        _mask_kernel,
        out_shape=[
            jax.ShapeDtypeStruct((B, IMG, IMG), jnp.float32),
            jax.ShapeDtypeStruct((B, 1), jnp.float32),
        ],
    )(dist, a, a.T)

    return (score, mask.reshape(B, IMG, IMG, 1))


# speedup vs baseline: 1.0152x; 1.0152x over previous
import numpy as np
import jax
import jax.numpy as jnp
from jax.experimental import pallas as pl

B, H, W, C = 4, 32, 32, 192
N = H * W
IMG = 512
SIGMA = 4.0
KS = 33
_NB = 32
_NSTREAM = 4

_HIGH = jax.lax.Precision.HIGHEST


def _resize_blur_matrix():
    out_size, in_size = IMG, H
    s = (np.arange(out_size, dtype=np.float64) + 0.5) * (in_size / out_size) - 0.5
    j = np.arange(in_size, dtype=np.float64)
    w = np.maximum(0.0, 1.0 - np.abs(s[None, :] - j[:, None]))
    w /= w.sum(axis=0, keepdims=True)
    R = w.T
    x = np.arange(KS, dtype=np.float64) - (KS - 1) / 2.0
    g = np.exp(-(x ** 2) / (2.0 * SIGMA * SIGMA))
    g /= g.sum()
    pad = KS // 2
    Bl = np.zeros((out_size, out_size), dtype=np.float64)
    for i in range(out_size):
        for t in range(KS):
            src = i - pad + t
            if src < 0:
                src = -src
            elif src >= out_size:
                src = 2 * out_size - 2 - src
            Bl[i, src] += g[t]
    return (Bl @ R).astype(np.float32)


_A = _resize_blur_matrix()


def _maha_kernel(f_ref, m_ref, *c_and_o):
    c_refs, o_ref = c_and_o[:-1], c_and_o[-1]
    delta = f_ref[...] - m_ref[...][:, None, :]
    d2s = []
    for s, c_ref in enumerate(c_refs):
        dl = delta[s * _NB:(s + 1) * _NB]
        y = jax.lax.dot_general(
            dl, c_ref[...],
            dimension_numbers=(((2,), (1,)), ((0,), (0,))),
            precision=jax.lax.Precision.DEFAULT,
        )
        d2s.append(jnp.sum(y * dl, axis=2))
    o_ref[...] = jnp.concatenate(d2s, axis=0)


def _mask_kernel(d_ref, a_ref, at_ref, mask_ref, score_ref):
    a = a_ref[...]
    at = at_ref[...]
    scores = []
    for b in range(B):
        d = jnp.sqrt(jnp.maximum(d_ref[b], 0.0))
        t = jax.lax.dot(a, d, precision=_HIGH)
        m = jax.lax.dot(t, at, precision=_HIGH)
        mask_ref[b] = m
        scores.append(jnp.max(m))
    score_ref[...] = jnp.stack(scores).reshape(B, 1)


def kernel(inputs, mean, cvar_inv):
    feature = inputs.reshape(B, N, C).transpose(1, 0, 2)
    blk = _NSTREAM * _NB

    d2 = pl.pallas_call(
        _maha_kernel,
        grid=(N // blk,),
        in_specs=[
            pl.BlockSpec((blk, B, C), lambda i: (i, 0, 0)),
            pl.BlockSpec((blk, C), lambda i: (i, 0)),
        ] + [
            pl.BlockSpec((_NB, C, C), lambda i, s=s: (i * _NSTREAM + s, 0, 0))
            for s in range(_NSTREAM)
        ],
        out_specs=pl.BlockSpec((blk, B), lambda i: (i, 0)),
        out_shape=jax.ShapeDtypeStruct((N, B), jnp.float32),
    )(feature, mean, *([cvar_inv] * _NSTREAM))

    d2g = d2.T.reshape(B, H, W)
    a = jnp.asarray(_A)
    mask, score = pl.pallas_call(
        _mask_kernel,
        out_shape=[
            jax.ShapeDtypeStruct((B, IMG, IMG), jnp.float32),
            jax.ShapeDtypeStruct((B, 1), jnp.float32),
        ],
    )(d2g, a, a.T)

    return (score, mask.reshape(B, IMG, IMG, 1))
